# baseline (speedup 1.0000x reference)
import functools

import jax
import jax.numpy as jnp
from jax import lax
from jax.experimental import pallas as pl
from jax.experimental.pallas import tpu as pltpu
from jax.experimental.pallas import tpu_sc as plsc

_BATCH = 16384
_DIM = 32
_SUPDIM = 128
_RPS = _SUPDIM // _DIM
_NUM_CORES = 2
_NUM_SUBCORES = 16
_NUM_WORKERS = _NUM_CORES * _NUM_SUBCORES
_B_PER_W = _BATCH // _NUM_WORKERS
_CHUNK = 128
_NCHUNK = _B_PER_W // _CHUNK


def _gather_body(idx_hbm, table_hbm, out_hbm, idx_v, sup_v, off_v, rows_v,
                 out_v, sem):
    wid = lax.axis_index("s") * _NUM_CORES + lax.axis_index("c")
    base = wid * _B_PER_W
    pltpu.sync_copy(idx_hbm.at[pl.ds(base, _B_PER_W)], idx_v)

    @plsc.parallel_loop(0, _B_PER_W, 16)
    def _(i):
        v = idx_v[pl.ds(i, 16)]
        sup_v[pl.ds(i, 16)] = v >> 2
        off_v[pl.ds(i, 16)] = (v & 3) * _DIM

    copies = []
    for j in range(_NCHUNK):
        copies.append(pltpu.async_copy(
            table_hbm.at[sup_v.at[pl.ds(j * _CHUNK, _CHUNK)]],
            rows_v.at[pl.ds(j * _CHUNK, _CHUNK), :],
            sem,
        ))
    for c in copies:
        c.wait()

    lanes = lax.iota(jnp.int32, 16)

    @plsc.parallel_loop(0, _B_PER_W * _DIM, 16, unroll=8)
    def _(e0):
        e = e0 + lanes
        jv = e >> 5
        cv = e & 31
        offv = plsc.load_gather(off_v, [jv])
        out_v[pl.ds(e0, 16)] = plsc.load_gather(rows_v, [jv, offv + cv])

    pltpu.sync_copy(out_v, out_hbm.at[pl.ds(base * _DIM, _B_PER_W * _DIM)])


@jax.jit
def kernel(x, embedding_publisher):
    idx = x[:, 0].astype(jnp.int32)
    table2 = embedding_publisher.reshape(-1, _SUPDIM)
    mesh = plsc.VectorSubcoreMesh(core_axis_name="c", subcore_axis_name="s")
    run = functools.partial(
        pl.kernel,
        mesh=mesh,
        out_type=jax.ShapeDtypeStruct((_BATCH * _DIM,), jnp.float32),
        scratch_types=[
            pltpu.VMEM((_B_PER_W,), jnp.int32),
            pltpu.VMEM((_B_PER_W,), jnp.int32),
            pltpu.VMEM((_B_PER_W,), jnp.int32),
            pltpu.VMEM((_B_PER_W, _SUPDIM), jnp.float32),
            pltpu.VMEM((_B_PER_W * _DIM,), jnp.float32),
            pltpu.SemaphoreType.DMA,
        ],
        compiler_params=pltpu.CompilerParams(needs_layout_passes=False),
    )(_gather_body)
    out_flat = run(idx, table2)
    return out_flat.reshape(_BATCH, _DIM)


# speedup vs baseline: 3.9063x; 3.9063x over previous
import functools

import jax
import jax.numpy as jnp
from jax import lax
from jax.experimental import pallas as pl
from jax.experimental.pallas import tpu as pltpu
from jax.experimental.pallas import tpu_sc as plsc

_BATCH = 16384
_DIM = 32
_LANES = 128
_NUM_CORES = 2
_NUM_SUBCORES = 16
_NUM_WORKERS = _NUM_CORES * _NUM_SUBCORES
_B_PER_W = _BATCH // _NUM_WORKERS
_CHUNK = 16
_NCHUNK = _B_PER_W // _CHUNK
_TC_PER_W = _B_PER_W // _LANES
_NTC = _BATCH // _LANES


def _gather_body(idx_hbm, tableT_hbm, out_hbm, idx_v, q_v, r_v, blocks_v,
                 out_v, sem):
    wid = lax.axis_index("s") * _NUM_CORES + lax.axis_index("c")
    base = wid * _B_PER_W
    pltpu.sync_copy(idx_hbm.at[pl.ds(base, _B_PER_W)], idx_v)

    @plsc.parallel_loop(0, _B_PER_W, 16)
    def _(i):
        v = idx_v[pl.ds(i, 16)]
        q_v[pl.ds(i, 16)] = v & jnp.int32(~(_LANES - 1))
        r_v[pl.ds(i, 16)] = v & jnp.int32(_LANES - 1)

    lanes = lax.iota(jnp.int32, 16)

    def chunk_body(g, carry):
        k0 = g * _CHUNK
        qv = q_v[pl.ds(k0, 16)]
        copies = []
        for j in range(_CHUNK):
            copies.append(pltpu.make_async_copy(
                tableT_hbm.at[
                    :, pl.ds(pl.multiple_of(qv[j], _LANES), _LANES)],
                blocks_v.at[j],
                sem,
            ))
        for c in copies:
            c.start()
        for c in copies:
            c.wait()

        rv = r_v[pl.ds(k0, 16)]
        tcl = k0 // _LANES
        kin = k0 % _LANES
        for c in range(_DIM):
            vals = plsc.load_gather(
                blocks_v, [lanes, jnp.full((16,), c, jnp.int32), rv])
            pos = ((c // 8) * _TC_PER_W + tcl) * 1024 + (c % 8) * 128 + kin
            out_v[pl.ds(pos, 16)] = vals
        return carry

    lax.fori_loop(0, _NCHUNK, chunk_body, 0)

    for tr in range(_DIM // 8):
        for tcl in range(_TC_PER_W):
            tc = wid * _TC_PER_W + tcl
            pltpu.sync_copy(
                out_v.at[pl.ds((tr * _TC_PER_W + tcl) * 1024, 1024)],
                out_hbm.at[pl.ds((tr * _NTC + tc) * 1024, 1024)],
            )


@jax.jit
def kernel(x, embedding_publisher):
    idx = x[:, 0].astype(jnp.int32)
    tableT = embedding_publisher.T
    mesh = plsc.VectorSubcoreMesh(core_axis_name="c", subcore_axis_name="s")
    run = functools.partial(
        pl.kernel,
        mesh=mesh,
        out_type=jax.ShapeDtypeStruct((_BATCH * _DIM,), jnp.float32),
        scratch_types=[
            pltpu.VMEM((_B_PER_W,), jnp.int32),
            pltpu.VMEM((_B_PER_W,), jnp.int32),
            pltpu.VMEM((_B_PER_W,), jnp.int32),
            pltpu.VMEM((_CHUNK, _DIM, _LANES), jnp.float32),
            pltpu.VMEM((_B_PER_W * _DIM,), jnp.float32),
            pltpu.SemaphoreType.DMA,
        ],
        compiler_params=pltpu.CompilerParams(needs_layout_passes=False),
    )(_gather_body)
    out_flat = run(idx, tableT)
    out = (out_flat.reshape(_DIM // 8, _NTC, 8, _LANES)
           .transpose(0, 2, 1, 3)
           .reshape(_DIM, _BATCH)
           .T)
    return out
